# Initial kernel scaffold; baseline (speedup 1.0000x reference)
import jax
import jax.numpy as jnp
from jax.experimental import pallas as pl


def kernel(x_p, x_v, x_len, P_table, V_table, W, b):
    raise NotImplementedError("write your pallas kernel here")


# speedup vs baseline: 12.6226x; 12.6226x over previous
import functools
import jax
import jax.numpy as jnp
from jax import lax
from jax.experimental import pallas as pl
from jax.experimental.pallas import tpu as pltpu
from jax.experimental.pallas import tpu_sc as plsc

B, L = 4096, 200
D = 128
OUT = 512
NC, NS = 2, 16
NW = NC * NS
RPW = B // NW
C0, C1 = 128, 72

_mesh = plsc.VectorSubcoreMesh(core_axis_name="c", subcore_axis_name="s")


@functools.partial(
    pl.kernel,
    out_type=jax.ShapeDtypeStruct((B, 2 * D), jnp.float32),
    mesh=_mesh,
    scratch_types=[
        pltpu.VMEM((L,), jnp.int32),
        pltpu.VMEM((L,), jnp.int32),
        pltpu.VMEM((L,), jnp.int32),
        pltpu.VMEM((L,), jnp.int32),
        pltpu.VMEM((L, D), jnp.float32),
        pltpu.VMEM((L, D), jnp.float32),
        pltpu.VMEM((L, D), jnp.float32),
        pltpu.VMEM((L, D), jnp.float32),
        pltpu.VMEM((2 * D,), jnp.float32),
        pltpu.SemaphoreType.DMA,
        pltpu.SemaphoreType.DMA,
    ],
)
def _sc_pool(xp_hbm, xv_hbm, ptab_hbm, vtab_hbm, out_hbm,
             ip0, ip1, iv0, iv1, rp0, rp1, rv0, rv1, stage, sem0, sem1):
    wid = lax.axis_index("s") * NC + lax.axis_index("c")
    base = wid * RPW

    def load_idx(buf_p, buf_v, row):
        pltpu.sync_copy(xp_hbm.at[row], buf_p)
        pltpu.sync_copy(xv_hbm.at[row], buf_v)

    def start_gather(buf_p, buf_v, rows_p, rows_v, sem):
        pltpu.async_copy(ptab_hbm.at[buf_p.at[pl.ds(0, C0)]],
                         rows_p.at[pl.ds(0, C0)], sem)
        pltpu.async_copy(ptab_hbm.at[buf_p.at[pl.ds(C0, C1)]],
                         rows_p.at[pl.ds(C0, C1)], sem)
        pltpu.async_copy(vtab_hbm.at[buf_v.at[pl.ds(0, C0)]],
                         rows_v.at[pl.ds(0, C0)], sem)
        pltpu.async_copy(vtab_hbm.at[buf_v.at[pl.ds(C0, C1)]],
                         rows_v.at[pl.ds(C0, C1)], sem)

    def wait_gather(rows_p, rows_v, sem):
        pltpu.make_async_copy(ptab_hbm.at[pl.ds(0, L)], rows_p, sem).wait()
        pltpu.make_async_copy(vtab_hbm.at[pl.ds(0, L)], rows_v, sem).wait()

    def accumulate(rows_p, rows_v, row):
        zero = jnp.zeros((16,), jnp.float32)

        def body(r, carry):
            new = []
            for c in range(8):
                new.append(carry[c] + rows_p[r, pl.ds(c * 16, 16)])
            for c in range(8):
                new.append(carry[8 + c] + rows_v[r, pl.ds(c * 16, 16)])
            return tuple(new)

        acc = lax.fori_loop(0, L, body, tuple(zero for _ in range(16)),
                            unroll=False)
        for c in range(16):
            stage[pl.ds(c * 16, 16)] = acc[c]
        pltpu.sync_copy(stage, out_hbm.at[row])

    load_idx(ip0, iv0, base)
    start_gather(ip0, iv0, rp0, rv0, sem0)
    load_idx(ip1, iv1, base + 1)

    def outer(g2, _):
        g = base + g2 * 2
        wait_gather(rp0, rv0, sem0)
        load_idx(ip0, iv0, g + 2)
        start_gather(ip1, iv1, rp1, rv1, sem1)
        accumulate(rp0, rv0, g)
        wait_gather(rp1, rv1, sem1)
        load_idx(ip1, iv1, g + 3)
        start_gather(ip0, iv0, rp0, rv0, sem0)
        accumulate(rp1, rv1, g + 1)
        return 0

    lax.fori_loop(0, RPW // 2 - 1, outer, 0, unroll=False)

    wait_gather(rp0, rv0, sem0)
    start_gather(ip1, iv1, rp1, rv1, sem1)
    accumulate(rp0, rv0, base + RPW - 2)
    wait_gather(rp1, rv1, sem1)
    accumulate(rp1, rv1, base + RPW - 1)


BM = 512


def _tc_head(h_ref, len_ref, w_ref, b_ref, o_ref):
    h = h_ref[:] / len_ref[:]
    r = jnp.dot(h, w_ref[:], preferred_element_type=jnp.float32) + b_ref[:]
    m = jnp.max(r, axis=1, keepdims=True)
    z = r - m
    s = jnp.sum(jnp.exp(z), axis=1, keepdims=True)
    o_ref[:] = z - jnp.log(s)


def kernel(x_p, x_v, x_len, P_table, V_table, W, b):
    sums = _sc_pool(x_p, x_v, P_table, V_table)
    lens = x_len.astype(jnp.float32)
    return pl.pallas_call(
        _tc_head,
        grid=(B // BM,),
        in_specs=[
            pl.BlockSpec((BM, 2 * D), lambda i: (i, 0)),
            pl.BlockSpec((BM, 1), lambda i: (i, 0)),
            pl.BlockSpec((2 * D, OUT), lambda i: (0, 0)),
            pl.BlockSpec((1, OUT), lambda i: (0, 0)),
        ],
        out_specs=pl.BlockSpec((BM, OUT), lambda i: (i, 0)),
        out_shape=jax.ShapeDtypeStruct((B, OUT), jnp.float32),
    )(sums, lens, W, b.reshape(1, OUT))


# speedup vs baseline: 16.4153x; 1.3005x over previous
import functools
import jax
import jax.numpy as jnp
from jax import lax
from jax.experimental import pallas as pl
from jax.experimental.pallas import tpu as pltpu
from jax.experimental.pallas import tpu_sc as plsc

B, L = 4096, 200
D = 128
OUT = 512
NC, NS = 2, 16
NW = NC * NS
RPW = B // NW
C0, C1 = 128, 72

_mesh = plsc.VectorSubcoreMesh(core_axis_name="c", subcore_axis_name="s")


@functools.partial(
    pl.kernel,
    out_type=jax.ShapeDtypeStruct((B, 2 * D), jnp.float32),
    mesh=_mesh,
    scratch_types=[
        pltpu.VMEM((L,), jnp.int32),
        pltpu.VMEM((L,), jnp.int32),
        pltpu.VMEM((L,), jnp.int32),
        pltpu.VMEM((L,), jnp.int32),
        pltpu.VMEM((L, D), jnp.float32),
        pltpu.VMEM((L, D), jnp.float32),
        pltpu.VMEM((L, D), jnp.float32),
        pltpu.VMEM((L, D), jnp.float32),
        pltpu.VMEM((2 * D,), jnp.float32),
        pltpu.VMEM((2 * D,), jnp.float32),
        pltpu.SemaphoreType.DMA,
        pltpu.SemaphoreType.DMA,
        pltpu.SemaphoreType.DMA,
        pltpu.SemaphoreType.DMA,
        pltpu.SemaphoreType.DMA,
        pltpu.SemaphoreType.DMA,
    ],
)
def _sc_pool(xp_hbm, xv_hbm, ptab_hbm, vtab_hbm, out_hbm,
             ip0, ip1, iv0, iv1, rp0, rp1, rv0, rv1, stage0, stage1,
             gsem0, gsem1, isem0, isem1, osem0, osem1):
    wid = lax.axis_index("s") * NC + lax.axis_index("c")
    base = wid * RPW

    ibufs = ((ip0, iv0, isem0), (ip1, iv1, isem1))
    rbufs = ((rp0, rv0, gsem0), (rp1, rv1, gsem1))
    obufs = ((stage0, osem0), (stage1, osem1))

    def start_idx(s, row):
        buf_p, buf_v, sem = ibufs[s]
        pltpu.async_copy(xp_hbm.at[row], buf_p, sem)
        pltpu.async_copy(xv_hbm.at[row], buf_v, sem)

    def wait_idx(s):
        buf_p, buf_v, sem = ibufs[s]
        pltpu.make_async_copy(xp_hbm.at[0], buf_p, sem).wait()
        pltpu.make_async_copy(xv_hbm.at[0], buf_v, sem).wait()

    def start_gather(s):
        buf_p, buf_v, _ = ibufs[s]
        rows_p, rows_v, sem = rbufs[s]
        pltpu.async_copy(ptab_hbm.at[buf_p.at[pl.ds(0, C0)]],
                         rows_p.at[pl.ds(0, C0)], sem)
        pltpu.async_copy(ptab_hbm.at[buf_p.at[pl.ds(C0, C1)]],
                         rows_p.at[pl.ds(C0, C1)], sem)
        pltpu.async_copy(vtab_hbm.at[buf_v.at[pl.ds(0, C0)]],
                         rows_v.at[pl.ds(0, C0)], sem)
        pltpu.async_copy(vtab_hbm.at[buf_v.at[pl.ds(C0, C1)]],
                         rows_v.at[pl.ds(C0, C1)], sem)

    def wait_gather(s):
        rows_p, rows_v, sem = rbufs[s]
        pltpu.make_async_copy(ptab_hbm.at[pl.ds(0, L)], rows_p, sem).wait()
        pltpu.make_async_copy(vtab_hbm.at[pl.ds(0, L)], rows_v, sem).wait()

    def wait_out(s):
        stage, sem = obufs[s]
        pltpu.make_async_copy(ptab_hbm.at[0], stage.at[pl.ds(0, D)], sem).wait()
        pltpu.make_async_copy(ptab_hbm.at[0], stage.at[pl.ds(D, D)], sem).wait()

    def accumulate_store(s, row):
        rows_p, rows_v, _ = rbufs[s]
        stage, sem = obufs[s]
        zero = jnp.zeros((16,), jnp.float32)

        def body(r, carry):
            new = []
            for c in range(8):
                new.append(carry[c] + rows_p[r, pl.ds(c * 16, 16)])
            for c in range(8):
                new.append(carry[8 + c] + rows_v[r, pl.ds(c * 16, 16)])
            return tuple(new)

        acc = lax.fori_loop(0, L, body, tuple(zero for _ in range(16)),
                            unroll=2)
        for c in range(16):
            stage[pl.ds(c * 16, 16)] = acc[c]
        pltpu.async_copy(stage, out_hbm.at[row], sem)

    def pair(g, drain_out):
        for b in (0, 1):
            s, o = b, 1 - b
            row = g + b
            wait_idx(o)
            start_gather(o)
            wait_gather(s)
            start_idx(s, row + 2)
            if drain_out:
                wait_out(s)
            accumulate_store(s, row)

    start_idx(0, base)
    start_idx(1, base + 1)
    wait_idx(0)
    start_gather(0)

    pair(base, drain_out=False)

    def outer(g2, _):
        pair(base + g2 * 2, drain_out=True)
        return 0

    lax.fori_loop(1, RPW // 2 - 1, outer, 0, unroll=False)

    wait_idx(1)
    start_gather(1)
    wait_gather(0)
    wait_out(0)
    accumulate_store(0, base + RPW - 2)
    wait_gather(1)
    wait_out(1)
    accumulate_store(1, base + RPW - 1)
    wait_out(0)
    wait_out(1)


BM = 512


def _tc_head(h_ref, len_ref, w_ref, b_ref, o_ref):
    h = h_ref[:] / len_ref[:]
    r = jnp.dot(h, w_ref[:], preferred_element_type=jnp.float32) + b_ref[:]
    m = jnp.max(r, axis=1, keepdims=True)
    z = r - m
    s = jnp.sum(jnp.exp(z), axis=1, keepdims=True)
    o_ref[:] = z - jnp.log(s)


def kernel(x_p, x_v, x_len, P_table, V_table, W, b):
    sums = _sc_pool(x_p, x_v, P_table, V_table)
    lens = x_len.astype(jnp.float32)
    return pl.pallas_call(
        _tc_head,
        grid=(B // BM,),
        in_specs=[
            pl.BlockSpec((BM, 2 * D), lambda i: (i, 0)),
            pl.BlockSpec((BM, 1), lambda i: (i, 0)),
            pl.BlockSpec((2 * D, OUT), lambda i: (0, 0)),
            pl.BlockSpec((1, OUT), lambda i: (0, 0)),
        ],
        out_specs=pl.BlockSpec((BM, OUT), lambda i: (i, 0)),
        out_shape=jax.ShapeDtypeStruct((B, OUT), jnp.float32),
    )(sums, lens, W, b.reshape(1, OUT))
